# baseline (speedup 1.0000x reference)
import functools

import jax
import jax.numpy as jnp
from jax import lax
from jax.experimental import pallas as pl
from jax.experimental.pallas import tpu as pltpu
from jax.experimental.pallas import tpu_sc as plsc

N = 50000
N2 = 50048
E = 800000
B = 1024
ATOM = 68
H = 16
NC, NS = 2, 16
NW = NC * NS
EPW = E // NW
C = 128
NCHK = 196
NBUF = 3
NB = 400
GRID = N // NB
NPT = N2 // NS
NCH = N2 // C
EPS = 1e-16

_mesh = plsc.VectorSubcoreMesh(
    core_axis_name="c", subcore_axis_name="s", num_cores=NC, num_subcores=NS)
_sc_params = pltpu.CompilerParams(
    needs_layout_passes=False, use_tc_tiling_on_sc=False)


def _prep_tail(h, as_ref, ad_ref, h_ref, s_ref, d_ref, mub_ref, mx, i):
    s = jnp.dot(h, as_ref[...], preferred_element_type=jnp.float32)
    d = jnp.dot(h, ad_ref[...], preferred_element_type=jnp.float32)
    h_ref[...] = h
    s_ref[...] = s
    d_ref[...] = d
    sm = jnp.max(s)
    dm = jnp.max(d)

    @pl.when(i == 0)
    def _():
        mx[0] = sm
        mx[1] = dm

    @pl.when(i > 0)
    def _():
        mx[0] = jnp.maximum(mx[0], sm)
        mx[1] = jnp.maximum(mx[1], dm)

    t = mx[0] + mx[1]
    mub_ref[...] = jnp.full((1, 16), jnp.where(t < 0, 0.2 * t, t), jnp.float32)


def _prep0_body(x_ref, w_ref, as_ref, ad_ref, h_ref, s_ref, d_ref, mub_ref, mx):
    h = jnp.dot(x_ref[...], w_ref[...], preferred_element_type=jnp.float32)
    _prep_tail(h, as_ref, ad_ref, h_ref, s_ref, d_ref, mub_ref, mx,
               pl.program_id(0))


_prep_outs = dict(
    out_specs=[
        pl.BlockSpec((NB, H), lambda i: (i, 0)),
        pl.BlockSpec((NB, 1), lambda i: (i, 0)),
        pl.BlockSpec((NB, 1), lambda i: (i, 0)),
        pl.BlockSpec((1, 16), lambda i: (0, 0)),
    ],
    out_shape=[
        jax.ShapeDtypeStruct((N, H), jnp.float32),
        jax.ShapeDtypeStruct((N, 1), jnp.float32),
        jax.ShapeDtypeStruct((N, 1), jnp.float32),
        jax.ShapeDtypeStruct((1, 16), jnp.float32),
    ],
    scratch_shapes=[pltpu.SMEM((2,), jnp.float32)],
)


def _prep0(x, W0, a0s, a0d):
    return pl.pallas_call(
        _prep0_body,
        grid=(GRID,),
        in_specs=[
            pl.BlockSpec((NB, ATOM), lambda i: (i, 0)),
            pl.BlockSpec((ATOM, H), lambda i: (0, 0)),
            pl.BlockSpec((H, 1), lambda i: (0, 0)),
            pl.BlockSpec((H, 1), lambda i: (0, 0)),
        ],
        **_prep_outs,
    )(x, W0, a0s, a0d)


def _combine(acc_ref, den_ref):
    a = acc_ref[0] + acc_ref[1]
    dent = jnp.sum(den_ref[...], axis=0)
    return jnp.maximum(a / (dent + EPS), 0.0)


def _prepl_body(acc_ref, den_ref, w_ref, as_ref, ad_ref,
                h_ref, s_ref, d_ref, mub_ref, mx):
    g = _combine(acc_ref, den_ref)
    h = jnp.dot(g, w_ref[...], preferred_element_type=jnp.float32)
    _prep_tail(h, as_ref, ad_ref, h_ref, s_ref, d_ref, mub_ref, mx,
               pl.program_id(0))


def _prepl(acc, den, Wl, als, ald):
    return pl.pallas_call(
        _prepl_body,
        grid=(GRID,),
        in_specs=[
            pl.BlockSpec((NC, NB, H), lambda i: (0, i, 0)),
            pl.BlockSpec((NW, NB, 1), lambda i: (0, i, 0)),
            pl.BlockSpec((H, H), lambda i: (0, 0)),
            pl.BlockSpec((H, 1), lambda i: (0, 0)),
            pl.BlockSpec((H, 1), lambda i: (0, 0)),
        ],
        **_prep_outs,
    )(acc, den, Wl, als, ald)


def _prepf_body(acc_ref, den_ref, u_ref):
    h2 = _combine(acc_ref, den_ref)
    u_ref[...] = jnp.concatenate(
        [h2, jnp.ones((NB, 1), jnp.float32), jnp.zeros((NB, 15), jnp.float32)],
        axis=1)


def _prepf(acc, den):
    return pl.pallas_call(
        _prepf_body,
        grid=(GRID,),
        in_specs=[
            pl.BlockSpec((NC, NB, H), lambda i: (0, i, 0)),
            pl.BlockSpec((NW, NB, 1), lambda i: (0, i, 0)),
        ],
        out_specs=pl.BlockSpec((NB, 32), lambda i: (i, 0)),
        out_shape=jax.ShapeDtypeStruct((N2, 32), jnp.float32),
    )(acc, den)


@functools.partial(
    pl.kernel,
    out_type=(jax.ShapeDtypeStruct((NC, N2, H), jnp.float32),
              jax.ShapeDtypeStruct((NW, N), jnp.float32)),
    mesh=_mesh,
    compiler_params=_sc_params,
    scratch_types=[
        pltpu.VMEM((16,), jnp.float32),
        pltpu.VMEM((N,), jnp.float32),
        [pltpu.VMEM((C,), jnp.int32)] * NBUF,
        [pltpu.VMEM((C,), jnp.int32)] * NBUF,
        [pltpu.VMEM((C,), jnp.float32)] * NBUF,
        [pltpu.VMEM((C,), jnp.float32)] * NBUF,
        [pltpu.VMEM((C, H), jnp.float32)] * NBUF,
        [pltpu.VMEM((C, H), jnp.float32)] * NBUF,
        [pltpu.SemaphoreType.DMA] * NBUF,
        [pltpu.SemaphoreType.DMA] * NBUF,
        [pltpu.SemaphoreType.DMA] * NBUF,
        [pltpu.SemaphoreType.DMA] * NBUF,
        [pltpu.SemaphoreType.DMA] * NBUF,
        [pltpu.SemaphoreType.DMA] * NBUF,
        pltpu.SemaphoreType.DMA,
        pltpu.VMEM_SHARED((N,), jnp.float32),
        pltpu.VMEM_SHARED((N,), jnp.float32),
        pltpu.VMEM_SHARED((N2, H), jnp.float32),
    ],
)
def _edge(h_hbm, s_hbm, d_hbm, mub_hbm, src_hbm, dst_hbm, acc_out, den_out,
          mub_v, den_v, srcb, dstb, sbuf, dbuf, rows, st,
          isa, isb, g1s, g2s, g3s, ssem, bsem, s_sh, d_sh, acc_sh):
    cid = lax.axis_index("c")
    sid = lax.axis_index("s")
    wid = cid * NS + sid
    iota = lax.broadcasted_iota(jnp.int32, (16,), 0)
    zv = jnp.zeros((16,), jnp.float32)
    c16 = jnp.full((16,), 16, jnp.int32)
    ebase = wid * EPW
    row0 = sid * NPT

    for b in range(NBUF):
        @pl.loop(0, C)
        def _(r, _b=b):
            st[_b][r, :] = zv

    @pl.loop(0, N // 16)
    def _(r):
        den_v[pl.ds(r * 16, 16)] = zv

    for j in range(24):
        pltpu.async_copy(st[0], acc_sh.at[pl.ds(row0 + j * C, C), :], bsem)
    for j in range(24):
        pltpu.make_async_copy(
            st[0], acc_sh.at[pl.ds(row0 + j * C, C), :], bsem).wait()
    pltpu.sync_copy(st[0].at[pl.ds(0, 56), :],
                    acc_sh.at[pl.ds(row0 + 24 * C, 56), :])

    @pl.when(sid < 10)
    def _():
        pltpu.sync_copy(s_hbm.at[pl.ds(sid * 5000, 5000)],
                        s_sh.at[pl.ds(sid * 5000, 5000)])

    @pl.when(sid >= 6)
    def _():
        pltpu.sync_copy(d_hbm.at[pl.ds((sid - 6) * 5000, 5000)],
                        d_sh.at[pl.ds((sid - 6) * 5000, 5000)])

    pltpu.sync_copy(mub_hbm, mub_v)
    mubv = mub_v[...]
    plsc.subcore_barrier()

    def issue_idx(ci, b):
        base = ebase + ci * C
        pltpu.async_copy(src_hbm.at[pl.ds(base, C)], srcb[b], isa[b])
        pltpu.async_copy(dst_hbm.at[pl.ds(base, C)], dstb[b], isb[b])

    def wait_idx(b):
        pltpu.make_async_copy(src_hbm.at[pl.ds(0, C)], srcb[b], isa[b]).wait()
        pltpu.make_async_copy(dst_hbm.at[pl.ds(0, C)], dstb[b], isb[b]).wait()

    def issue_gather(b):
        pltpu.async_copy(h_hbm.at[srcb[b]], rows[b], g1s[b])
        pltpu.async_copy(s_sh.at[srcb[b]], sbuf[b], g2s[b])
        pltpu.async_copy(d_sh.at[dstb[b]], dbuf[b], g3s[b])

    def wait_gather(b):
        pltpu.make_async_copy(h_hbm.at[srcb[b]], rows[b], g1s[b]).wait()
        pltpu.make_async_copy(s_sh.at[srcb[b]], sbuf[b], g2s[b]).wait()
        pltpu.make_async_copy(d_sh.at[dstb[b]], dbuf[b], g3s[b]).wait()

    def issue_scatter(b):
        pltpu.async_copy(st[b], acc_sh.at[dstb[b]], ssem[b], add=True)

    def wait_scatter(b):
        pltpu.make_async_copy(st[b], acc_sh.at[dstb[b]], ssem[b]).wait()

    def compute(ci, b):
        valid = EPW - ci * C

        @pl.loop(0, C // 16)
        def _(g):
            lanes = g * 16 + iota
            sv = sbuf[b][pl.ds(g * 16, 16)]
            dv = dbuf[b][pl.ds(g * 16, 16)]
            jv = dstb[b][pl.ds(g * 16, 16)]
            t = sv + dv
            e = jnp.where(t < 0.0, 0.2 * t, t)
            ex = jnp.exp(e - mubv)
            ex = jnp.where(lanes < valid, ex, 0.0)
            plsc.addupdate_scatter(den_v, [jv], ex)
            for f in range(16):
                fidx = jnp.full((16,), f, jnp.int32)
                hv = plsc.load_gather(rows[b], [lanes, fidx])
                plsc.store_scatter(st[b], [lanes, fidx], hv * ex)

    issue_idx(0, 0)
    wait_idx(0)
    issue_gather(0)
    issue_idx(1, 1)

    wait_idx(1)
    issue_gather(1)
    wait_gather(0)
    compute(0, 0)
    issue_scatter(0)
    issue_idx(2, 2)

    @pl.loop(1, NCHK - 3, step=NBUF)
    def _(co):
        for b in range(NBUF):
            ci = co + b
            s0 = (1 + b) % NBUF
            s1 = (2 + b) % NBUF
            s2 = b
            wait_idx(s1)
            issue_gather(s1)
            wait_gather(s0)
            compute(ci, s0)
            issue_scatter(s0)
            wait_scatter(s2)
            issue_idx(ci + 2, s2)

    wait_idx(2)
    issue_gather(2)
    wait_gather(1)
    compute(NCHK - 3, 1)
    issue_scatter(1)
    wait_scatter(0)
    issue_idx(NCHK - 1, 0)

    wait_idx(0)
    issue_gather(0)
    wait_gather(2)
    compute(NCHK - 2, 2)
    issue_scatter(2)
    wait_scatter(1)

    wait_gather(0)
    compute(NCHK - 1, 0)
    issue_scatter(0)
    wait_scatter(2)
    wait_scatter(0)

    plsc.subcore_barrier()

    for j in range(NPT // C + 1):
        rcnt = C if j < NPT // C else NPT - (NPT // C) * C
        pltpu.async_copy(acc_sh.at[pl.ds(row0 + j * C, rcnt), :],
                         acc_out.at[cid, pl.ds(row0 + j * C, rcnt), :], bsem)
    for j in range(NPT // C + 1):
        rcnt = C if j < NPT // C else NPT - (NPT // C) * C
        pltpu.make_async_copy(
            acc_sh.at[pl.ds(row0 + j * C, rcnt), :],
            acc_out.at[cid, pl.ds(row0 + j * C, rcnt), :], bsem).wait()

    pltpu.sync_copy(den_v, den_out.at[wid])


@functools.partial(
    pl.kernel,
    out_type=jax.ShapeDtypeStruct((NC, B, 32), jnp.float32),
    mesh=_mesh,
    compiler_params=_sc_params,
    scratch_types=[
        pltpu.VMEM((C, 32), jnp.float32),
        pltpu.VMEM((C,), jnp.int32),
        pltpu.VMEM((64, 32), jnp.float32),
        pltpu.VMEM_SHARED((B + 8, 32), jnp.float32),
    ],
)
def _pool(u_hbm, batch_hbm, out_hbm, buf, bb, zbuf, pool_sh):
    cid = lax.axis_index("c")
    sid = lax.axis_index("s")
    wid = cid * NS + sid
    zv = jnp.zeros((16,), jnp.float32)

    @pl.loop(0, 64)
    def _(r):
        zbuf[r, 0:16] = zv
        zbuf[r, 16:32] = zv

    pltpu.sync_copy(zbuf, pool_sh.at[pl.ds(sid * 64, 64), :])

    @pl.when(sid == 0)
    def _():
        pltpu.sync_copy(zbuf.at[pl.ds(0, 8), :], pool_sh.at[pl.ds(B, 8), :])

    plsc.subcore_barrier()

    @pl.loop(0, 13)
    def _(ci):
        chunk = wid + ci * NW

        @pl.when(chunk < NCH)
        def _():
            base = chunk * C
            pltpu.sync_copy(u_hbm.at[pl.ds(base, C), :], buf)
            pltpu.sync_copy(batch_hbm.at[pl.ds(base, C)], bb)
            pltpu.sync_copy(buf, pool_sh.at[bb], add=True)

    plsc.subcore_barrier()
    pltpu.sync_copy(pool_sh.at[pl.ds(sid * 64, 64), :],
                    out_hbm.at[cid, pl.ds(sid * 64, 64), :])


def _final_body(pool_ref, fp_ref, fw1_ref, fb1_ref, fw2_ref, fb2_ref,
                fcw_ref, fcb_ref, out_ref):
    p = pool_ref[0] + pool_ref[1]
    gnn = p[:, 0:16] / jnp.maximum(p[:, 16:17], 1.0)
    f1 = jnp.maximum(
        jnp.dot(fp_ref[...], fw1_ref[...], preferred_element_type=jnp.float32)
        + fb1_ref[...], 0.0)
    f2 = (jnp.dot(f1, fw2_ref[...], preferred_element_type=jnp.float32)
          + fb2_ref[...])
    cat = jnp.concatenate([gnn, f2], axis=1)
    out_ref[...] = (jnp.dot(cat, fcw_ref[...], preferred_element_type=jnp.float32)
                    + fcb_ref[...])


def _final(pool, fp, fW1, fb1, fW2, fb2, fcW, fcb):
    return pl.pallas_call(
        _final_body,
        out_shape=jax.ShapeDtypeStruct((B, 1), jnp.float32),
    )(pool, fp, fW1, fb1, fW2, fb2, fcW, fcb)


def kernel(x, edge_index, fp, batch, W0, a0s, a0d, W1, a1s, a1d, W2, a2s, a2d,
           fW1, fb1, fW2, fb2, fcW, fcb):
    src = edge_index[0]
    dst = edge_index[1]
    W1e = W1[0:16] + W1[16:32] + W1[32:48] + W1[48:64]
    W2e = W2[0:16] + W2[16:32] + W2[32:48] + W2[48:64]

    pad = jnp.zeros((NW * NCHK * C - E,), jnp.int32)
    src_p = jnp.concatenate([src, pad])
    dst_p = jnp.concatenate([dst, pad])

    h, s, d, mub = _prep0(x, W0, a0s.reshape(H, 1), a0d.reshape(H, 1))
    acc, den = _edge(h, s.reshape(N), d.reshape(N), mub.reshape(16),
                     src_p, dst_p)
    h, s, d, mub = _prepl(acc, den.reshape(NW, N, 1), W1e,
                          a1s.reshape(H, 1), a1d.reshape(H, 1))
    acc, den = _edge(h, s.reshape(N), d.reshape(N), mub.reshape(16),
                     src_p, dst_p)
    h, s, d, mub = _prepl(acc, den.reshape(NW, N, 1), W2e,
                          a2s.reshape(H, 1), a2d.reshape(H, 1))
    acc, den = _edge(h, s.reshape(N), d.reshape(N), mub.reshape(16),
                     src_p, dst_p)

    h2ext = _prepf(acc, den.reshape(NW, N, 1))
    batch_pad = jnp.concatenate([batch, jnp.full((N2 - N,), B, jnp.int32)])
    pool = _pool(h2ext, batch_pad)
    out = _final(pool, fp, fW1, fb1.reshape(1, 64), fW2, fb2.reshape(1, 16),
                 fcW, fcb.reshape(1, 1))
    return out.reshape(B)


# speedup vs baseline: 2.0254x; 2.0254x over previous
import functools

import jax
import jax.numpy as jnp
from jax import lax
from jax.experimental import pallas as pl
from jax.experimental.pallas import tpu as pltpu
from jax.experimental.pallas import tpu_sc as plsc

N = 50000
N2 = 50048
E = 800000
B = 1024
ATOM = 68
H = 16
NC, NS = 2, 16
NW = NC * NS
EPW = E // NW
C = 128
NCHK = 196
NBUF = 3
NB = 400
GRID = N // NB
NPT = N2 // NS
NCH = N2 // C
EPS = 1e-16

_mesh = plsc.VectorSubcoreMesh(
    core_axis_name="c", subcore_axis_name="s", num_cores=NC, num_subcores=NS)
_sc_params = pltpu.CompilerParams(
    needs_layout_passes=False, use_tc_tiling_on_sc=False)


def _prep_tail(h, as_ref, ad_ref, h_ref, s_ref, d_ref, mub_ref, mx, i):
    s = jnp.dot(h, as_ref[...], preferred_element_type=jnp.float32)
    d = jnp.dot(h, ad_ref[...], preferred_element_type=jnp.float32)
    h_ref[...] = h
    s_ref[...] = s
    d_ref[...] = d
    sm = jnp.max(s)
    dm = jnp.max(d)

    @pl.when(i == 0)
    def _():
        mx[0] = sm
        mx[1] = dm

    @pl.when(i > 0)
    def _():
        mx[0] = jnp.maximum(mx[0], sm)
        mx[1] = jnp.maximum(mx[1], dm)

    t = mx[0] + mx[1]
    mub_ref[...] = jnp.full((1, 16), jnp.where(t < 0, 0.2 * t, t), jnp.float32)


def _prep0_body(x_ref, w_ref, as_ref, ad_ref, h_ref, s_ref, d_ref, mub_ref, mx):
    h = jnp.dot(x_ref[...], w_ref[...], preferred_element_type=jnp.float32)
    _prep_tail(h, as_ref, ad_ref, h_ref, s_ref, d_ref, mub_ref, mx,
               pl.program_id(0))


_prep_outs = dict(
    out_specs=[
        pl.BlockSpec((NB, H), lambda i: (i, 0)),
        pl.BlockSpec((NB, 1), lambda i: (i, 0)),
        pl.BlockSpec((NB, 1), lambda i: (i, 0)),
        pl.BlockSpec((1, 16), lambda i: (0, 0)),
    ],
    out_shape=[
        jax.ShapeDtypeStruct((N, H), jnp.float32),
        jax.ShapeDtypeStruct((N, 1), jnp.float32),
        jax.ShapeDtypeStruct((N, 1), jnp.float32),
        jax.ShapeDtypeStruct((1, 16), jnp.float32),
    ],
    scratch_shapes=[pltpu.SMEM((2,), jnp.float32)],
)


def _prep0(x, W0, a0s, a0d):
    return pl.pallas_call(
        _prep0_body,
        grid=(GRID,),
        in_specs=[
            pl.BlockSpec((NB, ATOM), lambda i: (i, 0)),
            pl.BlockSpec((ATOM, H), lambda i: (0, 0)),
            pl.BlockSpec((H, 1), lambda i: (0, 0)),
            pl.BlockSpec((H, 1), lambda i: (0, 0)),
        ],
        **_prep_outs,
    )(x, W0, a0s, a0d)


def _combine(acc_ref):
    a = acc_ref[0] + acc_ref[1]
    den = a[:, 16:17]
    return jnp.maximum(a[:, 0:16] / (den + EPS), 0.0)


def _prepl_body(acc_ref, w_ref, as_ref, ad_ref,
                h_ref, s_ref, d_ref, mub_ref, mx):
    g = _combine(acc_ref)
    h = jnp.dot(g, w_ref[...], preferred_element_type=jnp.float32)
    _prep_tail(h, as_ref, ad_ref, h_ref, s_ref, d_ref, mub_ref, mx,
               pl.program_id(0))


def _prepl(acc, Wl, als, ald):
    return pl.pallas_call(
        _prepl_body,
        grid=(GRID,),
        in_specs=[
            pl.BlockSpec((NC, NB, 32), lambda i: (0, i, 0)),
            pl.BlockSpec((H, H), lambda i: (0, 0)),
            pl.BlockSpec((H, 1), lambda i: (0, 0)),
            pl.BlockSpec((H, 1), lambda i: (0, 0)),
        ],
        **_prep_outs,
    )(acc, Wl, als, ald)


def _prepf_body(acc_ref, u_ref):
    h2 = _combine(acc_ref)
    u_ref[...] = jnp.concatenate(
        [h2, jnp.ones((NB, 1), jnp.float32), jnp.zeros((NB, 15), jnp.float32)],
        axis=1)


def _prepf(acc):
    return pl.pallas_call(
        _prepf_body,
        grid=(GRID,),
        in_specs=[pl.BlockSpec((NC, NB, 32), lambda i: (0, i, 0))],
        out_specs=pl.BlockSpec((NB, 32), lambda i: (i, 0)),
        out_shape=jax.ShapeDtypeStruct((N2, 32), jnp.float32),
    )(acc)


@functools.partial(
    pl.kernel,
    out_type=jax.ShapeDtypeStruct((NC, N2, 32), jnp.float32),
    mesh=_mesh,
    compiler_params=_sc_params,
    scratch_types=[
        pltpu.VMEM((16,), jnp.float32),
        [pltpu.VMEM((C,), jnp.int32)] * NBUF,
        [pltpu.VMEM((C,), jnp.int32)] * NBUF,
        [pltpu.VMEM((C,), jnp.float32)] * NBUF,
        [pltpu.VMEM((C,), jnp.float32)] * NBUF,
        [pltpu.VMEM((C, H), jnp.float32)] * NBUF,
        [pltpu.VMEM((C, 32), jnp.float32)] * NBUF,
        [pltpu.SemaphoreType.DMA] * NBUF,
        [pltpu.SemaphoreType.DMA] * NBUF,
        [pltpu.SemaphoreType.DMA] * NBUF,
        [pltpu.SemaphoreType.DMA] * NBUF,
        [pltpu.SemaphoreType.DMA] * NBUF,
        [pltpu.SemaphoreType.DMA] * NBUF,
        pltpu.SemaphoreType.DMA,
        pltpu.VMEM_SHARED((N,), jnp.float32),
        pltpu.VMEM_SHARED((N,), jnp.float32),
        pltpu.VMEM_SHARED((N2, 32), jnp.float32),
    ],
)
def _edge(h_hbm, s_hbm, d_hbm, mub_hbm, src_hbm, dst_hbm, acc_out,
          mub_v, srcb, dstb, sbuf, dbuf, rows, st,
          isa, isb, g1s, g2s, g3s, ssem, bsem, s_sh, d_sh, acc_sh):
    cid = lax.axis_index("c")
    sid = lax.axis_index("s")
    wid = cid * NS + sid
    iota = lax.broadcasted_iota(jnp.int32, (16,), 0)
    zv = jnp.zeros((16,), jnp.float32)
    c16 = jnp.full((16,), 16, jnp.int32)
    ebase = wid * EPW
    row0 = sid * NPT

    for b in range(NBUF):
        @pl.loop(0, C)
        def _(r, _b=b):
            st[_b][r, 0:16] = zv
            st[_b][r, 16:32] = zv

    for j in range(24):
        pltpu.async_copy(st[0], acc_sh.at[pl.ds(row0 + j * C, C), :], bsem)
    for j in range(24):
        pltpu.make_async_copy(
            st[0], acc_sh.at[pl.ds(row0 + j * C, C), :], bsem).wait()
    pltpu.sync_copy(st[0].at[pl.ds(0, 56), :],
                    acc_sh.at[pl.ds(row0 + 24 * C, 56), :])

    @pl.when(sid < 10)
    def _():
        pltpu.sync_copy(s_hbm.at[pl.ds(sid * 5000, 5000)],
                        s_sh.at[pl.ds(sid * 5000, 5000)])

    @pl.when(sid >= 6)
    def _():
        pltpu.sync_copy(d_hbm.at[pl.ds((sid - 6) * 5000, 5000)],
                        d_sh.at[pl.ds((sid - 6) * 5000, 5000)])

    pltpu.sync_copy(mub_hbm, mub_v)
    mubv = mub_v[...]
    plsc.subcore_barrier()

    def issue_idx(ci, b):
        base = ebase + ci * C
        pltpu.async_copy(src_hbm.at[pl.ds(base, C)], srcb[b], isa[b])
        pltpu.async_copy(dst_hbm.at[pl.ds(base, C)], dstb[b], isb[b])

    def wait_idx(b):
        pltpu.make_async_copy(src_hbm.at[pl.ds(0, C)], srcb[b], isa[b]).wait()
        pltpu.make_async_copy(dst_hbm.at[pl.ds(0, C)], dstb[b], isb[b]).wait()

    def issue_gather(b):
        pltpu.async_copy(h_hbm.at[srcb[b]], rows[b], g1s[b])
        pltpu.async_copy(s_sh.at[srcb[b]], sbuf[b], g2s[b])
        pltpu.async_copy(d_sh.at[dstb[b]], dbuf[b], g3s[b])

    def wait_gather(b):
        pltpu.make_async_copy(h_hbm.at[srcb[b]], rows[b], g1s[b]).wait()
        pltpu.make_async_copy(s_sh.at[srcb[b]], sbuf[b], g2s[b]).wait()
        pltpu.make_async_copy(d_sh.at[dstb[b]], dbuf[b], g3s[b]).wait()

    def issue_scatter(b):
        pltpu.async_copy(st[b], acc_sh.at[dstb[b]], ssem[b], add=True)

    def wait_scatter(b):
        pltpu.make_async_copy(st[b], acc_sh.at[dstb[b]], ssem[b]).wait()

    def compute(ci, b):
        valid = EPW - ci * C

        @pl.loop(0, C // 16)
        def _(g):
            lanes = g * 16 + iota
            sv = sbuf[b][pl.ds(g * 16, 16)]
            dv = dbuf[b][pl.ds(g * 16, 16)]
            t = sv + dv
            e = jnp.where(t < 0.0, 0.2 * t, t)
            ex = jnp.exp(e - mubv)
            ex = jnp.where(lanes < valid, ex, 0.0)
            plsc.store_scatter(st[b], [lanes, c16], ex)
            for f in range(16):
                fidx = jnp.full((16,), f, jnp.int32)
                hv = plsc.load_gather(rows[b], [lanes, fidx])
                plsc.store_scatter(st[b], [lanes, fidx], hv * ex)

    issue_idx(0, 0)
    wait_idx(0)
    issue_gather(0)
    issue_idx(1, 1)

    wait_idx(1)
    issue_gather(1)
    wait_gather(0)
    compute(0, 0)
    issue_scatter(0)
    issue_idx(2, 2)

    @pl.loop(1, NCHK - 3, step=NBUF)
    def _(co):
        for b in range(NBUF):
            ci = co + b
            s0 = (1 + b) % NBUF
            s1 = (2 + b) % NBUF
            s2 = b
            wait_idx(s1)
            issue_gather(s1)
            wait_gather(s0)
            compute(ci, s0)
            issue_scatter(s0)
            wait_scatter(s2)
            issue_idx(ci + 2, s2)

    wait_idx(2)
    issue_gather(2)
    wait_gather(1)
    compute(NCHK - 3, 1)
    issue_scatter(1)
    wait_scatter(0)
    issue_idx(NCHK - 1, 0)

    wait_idx(0)
    issue_gather(0)
    wait_gather(2)
    compute(NCHK - 2, 2)
    issue_scatter(2)
    wait_scatter(1)

    wait_gather(0)
    compute(NCHK - 1, 0)
    issue_scatter(0)
    wait_scatter(2)
    wait_scatter(0)

    plsc.subcore_barrier()

    for j in range(NPT // C + 1):
        rcnt = C if j < NPT // C else NPT - (NPT // C) * C
        pltpu.async_copy(acc_sh.at[pl.ds(row0 + j * C, rcnt), :],
                         acc_out.at[cid, pl.ds(row0 + j * C, rcnt), :], bsem)
    for j in range(NPT // C + 1):
        rcnt = C if j < NPT // C else NPT - (NPT // C) * C
        pltpu.make_async_copy(
            acc_sh.at[pl.ds(row0 + j * C, rcnt), :],
            acc_out.at[cid, pl.ds(row0 + j * C, rcnt), :], bsem).wait()


@functools.partial(
    pl.kernel,
    out_type=jax.ShapeDtypeStruct((NC, B, 32), jnp.float32),
    mesh=_mesh,
    compiler_params=_sc_params,
    scratch_types=[
        pltpu.VMEM((C, 32), jnp.float32),
        pltpu.VMEM((C,), jnp.int32),
        pltpu.VMEM((64, 32), jnp.float32),
        pltpu.VMEM_SHARED((B + 8, 32), jnp.float32),
    ],
)
def _pool(u_hbm, batch_hbm, out_hbm, buf, bb, zbuf, pool_sh):
    cid = lax.axis_index("c")
    sid = lax.axis_index("s")
    wid = cid * NS + sid
    zv = jnp.zeros((16,), jnp.float32)

    @pl.loop(0, 64)
    def _(r):
        zbuf[r, 0:16] = zv
        zbuf[r, 16:32] = zv

    pltpu.sync_copy(zbuf, pool_sh.at[pl.ds(sid * 64, 64), :])

    @pl.when(sid == 0)
    def _():
        pltpu.sync_copy(zbuf.at[pl.ds(0, 8), :], pool_sh.at[pl.ds(B, 8), :])

    plsc.subcore_barrier()

    @pl.loop(0, 13)
    def _(ci):
        chunk = wid + ci * NW

        @pl.when(chunk < NCH)
        def _():
            base = chunk * C
            pltpu.sync_copy(u_hbm.at[pl.ds(base, C), :], buf)
            pltpu.sync_copy(batch_hbm.at[pl.ds(base, C)], bb)
            pltpu.sync_copy(buf, pool_sh.at[bb], add=True)

    plsc.subcore_barrier()
    pltpu.sync_copy(pool_sh.at[pl.ds(sid * 64, 64), :],
                    out_hbm.at[cid, pl.ds(sid * 64, 64), :])


def _final_body(pool_ref, fp_ref, fw1_ref, fb1_ref, fw2_ref, fb2_ref,
                fcw_ref, fcb_ref, out_ref):
    p = pool_ref[0] + pool_ref[1]
    gnn = p[:, 0:16] / jnp.maximum(p[:, 16:17], 1.0)
    f1 = jnp.maximum(
        jnp.dot(fp_ref[...], fw1_ref[...], preferred_element_type=jnp.float32)
        + fb1_ref[...], 0.0)
    f2 = (jnp.dot(f1, fw2_ref[...], preferred_element_type=jnp.float32)
          + fb2_ref[...])
    cat = jnp.concatenate([gnn, f2], axis=1)
    out_ref[...] = (jnp.dot(cat, fcw_ref[...], preferred_element_type=jnp.float32)
                    + fcb_ref[...])


def _final(pool, fp, fW1, fb1, fW2, fb2, fcW, fcb):
    return pl.pallas_call(
        _final_body,
        out_shape=jax.ShapeDtypeStruct((B, 1), jnp.float32),
    )(pool, fp, fW1, fb1, fW2, fb2, fcW, fcb)


def kernel(x, edge_index, fp, batch, W0, a0s, a0d, W1, a1s, a1d, W2, a2s, a2d,
           fW1, fb1, fW2, fb2, fcW, fcb):
    src = edge_index[0]
    dst = edge_index[1]
    W1e = W1[0:16] + W1[16:32] + W1[32:48] + W1[48:64]
    W2e = W2[0:16] + W2[16:32] + W2[32:48] + W2[48:64]

    pad = jnp.zeros((NW * NCHK * C - E,), jnp.int32)
    src_p = jnp.concatenate([src, pad])
    dst_p = jnp.concatenate([dst, pad])

    h, s, d, mub = _prep0(x, W0, a0s.reshape(H, 1), a0d.reshape(H, 1))
    acc = _edge(h, s.reshape(N), d.reshape(N), mub.reshape(16), src_p, dst_p)
    h, s, d, mub = _prepl(acc, W1e, a1s.reshape(H, 1), a1d.reshape(H, 1))
    acc = _edge(h, s.reshape(N), d.reshape(N), mub.reshape(16), src_p, dst_p)
    h, s, d, mub = _prepl(acc, W2e, a2s.reshape(H, 1), a2d.reshape(H, 1))
    acc = _edge(h, s.reshape(N), d.reshape(N), mub.reshape(16), src_p, dst_p)

    h2ext = _prepf(acc)
    batch_pad = jnp.concatenate([batch, jnp.full((N2 - N,), B, jnp.int32)])
    pool = _pool(h2ext, batch_pad)
    out = _final(pool, fp, fW1, fb1.reshape(1, 64), fW2, fb2.reshape(1, 16),
                 fcW, fcb.reshape(1, 1))
    return out.reshape(B)
